# Initial kernel scaffold; baseline (speedup 1.0000x reference)
import jax
import jax.numpy as jnp
from jax.experimental import pallas as pl


def kernel(src_pts, tgt_pts, R_init, t_init, W1, b1, W2, b2, W_wl, Wd1, bd1, Wd2, bd2):
    raise NotImplementedError("write your pallas kernel here")


# speedup vs baseline: 12.8899x; 12.8899x over previous
import functools

import jax
import jax.numpy as jnp
from jax import lax
from jax.experimental import pallas as pl
from jax.experimental.pallas import tpu as pltpu
from jax.experimental.pallas import tpu_sc as plsc

B = 4
N = 8192
K_KEY = 64
NS_SRC = 32
NS_TGT = 16
C_VOX = 27
D_FEAT = 32
D_DFE = 64


def _k1_body(x_ref, w1_ref, b1_ref, w2_ref, b2_ref, wl_ref, a_ref, f_ref,
             table_ref, score_ref):
    x = x_ref[0]
    h = jnp.maximum(jnp.dot(x, w1_ref[...], preferred_element_type=jnp.float32)
                    + b1_ref[...], 0.0)
    feat = jnp.maximum(jnp.dot(h, w2_ref[...], preferred_element_type=jnp.float32)
                       + b2_ref[...], 0.0)
    score_ref[0] = jnp.dot(feat, wl_ref[...], preferred_element_type=jnp.float32)
    table_ref[0] = (jnp.dot(x, a_ref[...], preferred_element_type=jnp.float32)
                    + jnp.dot(feat, f_ref[...], preferred_element_type=jnp.float32))


def _run_k1(x_all, W1, b1, W2, b2, W_wl, A, Ff):
    nb = x_all.shape[0]
    return pl.pallas_call(
        _k1_body,
        grid=(nb,),
        in_specs=[
            pl.BlockSpec((1, N, 8), lambda i: (i, 0, 0)),
            pl.BlockSpec((8, D_FEAT), lambda i: (0, 0)),
            pl.BlockSpec((1, D_FEAT), lambda i: (0, 0)),
            pl.BlockSpec((D_FEAT, D_FEAT), lambda i: (0, 0)),
            pl.BlockSpec((1, D_FEAT), lambda i: (0, 0)),
            pl.BlockSpec((D_FEAT, 1), lambda i: (0, 0)),
            pl.BlockSpec((8, D_DFE), lambda i: (0, 0)),
            pl.BlockSpec((D_FEAT, D_DFE), lambda i: (0, 0)),
        ],
        out_specs=[
            pl.BlockSpec((1, N, D_DFE), lambda i: (i, 0, 0)),
            pl.BlockSpec((1, N, 1), lambda i: (i, 0, 0)),
        ],
        out_shape=[
            jax.ShapeDtypeStruct((nb, N, D_DFE), jnp.float32),
            jax.ShapeDtypeStruct((nb, N, 1), jnp.float32),
        ],
    )(x_all, W1, b1, W2, b2, W_wl, A, Ff)


def _k2_body(score_ref, xyz_ref, rt_ref, t_ref, offs_ref, a_ref, bd1_ref,
             key_ref, cand_ref, subs_ref, subt_ref):
    s = score_ref[0]
    iota = (lax.broadcasted_iota(jnp.int32, (64, 128), 0) * 128
            + lax.broadcasted_iota(jnp.int32, (64, 128), 1))

    def body(t, s):
        m = jnp.max(s)
        j = jnp.min(jnp.where(s == m, iota, jnp.int32(2**30)))
        key_ref[0, pl.ds(t, 1), :] = xyz_ref[0, pl.ds(j, 1), :]
        return jnp.where(iota == j, -jnp.inf, s)

    lax.fori_loop(0, K_KEY, body, s)

    key = key_ref[0]
    trans = (jnp.dot(key, rt_ref[...], preferred_element_type=jnp.float32)
             + t_ref[...])
    cand = (trans[:, None, :] + offs_ref[...][None, :, :]).reshape(
        K_KEY * C_VOX, 8)
    cand_ref[0] = cand
    subs_ref[0] = (jnp.dot(key, a_ref[...], preferred_element_type=jnp.float32)
                   - bd1_ref[...])
    subt_ref[0] = (jnp.dot(cand, a_ref[...], preferred_element_type=jnp.float32)
                   - bd1_ref[...])


def _run_k2(scores2, xyz_src, Rt, tT, offs, A, bd1):
    return pl.pallas_call(
        _k2_body,
        grid=(B,),
        in_specs=[
            pl.BlockSpec((1, 64, 128), lambda i: (i, 0, 0)),
            pl.BlockSpec((1, N, 8), lambda i: (i, 0, 0)),
            pl.BlockSpec((8, 8), lambda i: (0, 0)),
            pl.BlockSpec((1, 8), lambda i: (0, 0)),
            pl.BlockSpec((C_VOX, 8), lambda i: (0, 0)),
            pl.BlockSpec((8, D_DFE), lambda i: (0, 0)),
            pl.BlockSpec((1, D_DFE), lambda i: (0, 0)),
        ],
        out_specs=[
            pl.BlockSpec((1, K_KEY, 8), lambda i: (i, 0, 0)),
            pl.BlockSpec((1, K_KEY * C_VOX, 8), lambda i: (i, 0, 0)),
            pl.BlockSpec((1, K_KEY, D_DFE), lambda i: (i, 0, 0)),
            pl.BlockSpec((1, K_KEY * C_VOX, D_DFE), lambda i: (i, 0, 0)),
        ],
        out_shape=[
            jax.ShapeDtypeStruct((B, K_KEY, 8), jnp.float32),
            jax.ShapeDtypeStruct((B, K_KEY * C_VOX, 8), jnp.float32),
            jax.ShapeDtypeStruct((B, K_KEY, D_DFE), jnp.float32),
            jax.ShapeDtypeStruct((B, K_KEY * C_VOX, D_DFE), jnp.float32),
        ],
    )(scores2, xyz_src, Rt, tT, offs, A, bd1)


def _knn_body(q_ref, pt_ref, out_ref, *, k, rows):
    b = pl.program_id(0)
    q = q_ref[0]
    px = pt_ref[0]
    qx = q[:, 0:1]
    qy = q[:, 1:2]
    qz = q[:, 2:3]
    aa = qx * qx + qy * qy + qz * qz
    bb = jnp.sum(px * px, axis=0, keepdims=True)
    ab = (qx * px[0:1, :] + qy * px[1:2, :] + qz * px[2:3, :])
    dist = aa + bb - 2.0 * ab

    col = lax.broadcasted_iota(jnp.int32, (rows, N), 1)
    ocol = lax.broadcasted_iota(jnp.int32, (rows, k), 1)
    big = jnp.int32(2**30)

    def body(t, carry):
        d, out = carry
        m = jnp.min(d, axis=1, keepdims=True)
        cidx = jnp.where(d == m, col, big)
        j = jnp.min(cidx, axis=1, keepdims=True)
        d = jnp.where(cidx == j, jnp.inf, d)
        out = jnp.where(ocol == t, j + b * N, out)
        return d, out

    out0 = jnp.zeros((rows, k), jnp.int32)
    _, out = lax.fori_loop(0, k, body, (dist, out0))
    out_ref[0] = out


def _run_knn(q, ptsT, k, rows, nblk):
    nq = q.shape[1]
    body = functools.partial(_knn_body, k=k, rows=rows)
    return pl.pallas_call(
        body,
        grid=(B, nblk),
        in_specs=[
            pl.BlockSpec((1, rows, 8), lambda i, r: (i, r, 0)),
            pl.BlockSpec((1, 3, N), lambda i, r: (i, 0, 0)),
        ],
        out_specs=pl.BlockSpec((1, rows, k), lambda i, r: (i, r, 0)),
        out_shape=jax.ShapeDtypeStruct((B, nq, k), jnp.int32),
    )(q, ptsT)


_SC_CHUNK = 128


def _sc_gather_body(tsrc_ref, ttgt_ref, sidx_ref, tidx_ref, gsrc_ref,
                    gtgt_ref, idx_v, rows_v, sem, *, src_chunks, tgt_chunks):
    nc = 2
    wid = lax.axis_index("s") * nc + lax.axis_index("c")

    def run(table_ref, iref, oref, nchunk, base):
        def body(i, _):
            off = base + i * _SC_CHUNK
            pltpu.sync_copy(iref.at[pl.ds(off, _SC_CHUNK)], idx_v)
            pltpu.async_copy(table_ref.at[idx_v], rows_v, sem).wait()
            pltpu.sync_copy(rows_v, oref.at[pl.ds(off, _SC_CHUNK)])
            return 0

        lax.fori_loop(0, nchunk, body, 0)

    run(tsrc_ref, sidx_ref, gsrc_ref, src_chunks, wid * (src_chunks * _SC_CHUNK))
    run(ttgt_ref, tidx_ref, gtgt_ref, tgt_chunks, wid * (tgt_chunks * _SC_CHUNK))


def _run_sc_gather(table_src, table_tgt, sidx, tidx):
    ns_rows = sidx.shape[0]
    nt_rows = tidx.shape[0]
    nw = 32
    src_chunks = ns_rows // (nw * _SC_CHUNK)
    tgt_chunks = nt_rows // (nw * _SC_CHUNK)
    mesh = plsc.VectorSubcoreMesh(core_axis_name="c", subcore_axis_name="s")
    body = functools.partial(_sc_gather_body, src_chunks=src_chunks,
                             tgt_chunks=tgt_chunks)
    return pl.kernel(
        body,
        out_type=[
            jax.ShapeDtypeStruct((ns_rows, D_DFE), jnp.float32),
            jax.ShapeDtypeStruct((nt_rows, D_DFE), jnp.float32),
        ],
        mesh=mesh,
        scratch_types=[
            pltpu.VMEM((_SC_CHUNK,), jnp.int32),
            pltpu.VMEM((_SC_CHUNK, D_DFE), jnp.float32),
            pltpu.SemaphoreType.DMA,
        ],
        compiler_params=pltpu.CompilerParams(use_tc_tiling_on_sc=False),
    )(table_src, table_tgt, sidx, tidx)


def _dfe_tgt_body(g_ref, sub_ref, w2_ref, b2_ref, out_ref, *, rows):
    g = g_ref[0]
    sub = sub_ref[0]
    h1 = jnp.maximum(g.reshape(rows, NS_TGT, D_DFE) - sub[:, None, :], 0.0)
    h1 = h1.reshape(rows * NS_TGT, D_DFE)
    h2 = jnp.maximum(jnp.dot(h1, w2_ref[...], preferred_element_type=jnp.float32)
                     + b2_ref[...], 0.0)
    out_ref[0] = jnp.max(h2.reshape(rows, NS_TGT, D_DFE), axis=1)


def _run_dfe_tgt(g_tgt, sub_tgt, Wd2, bd2, rows, nblk):
    nq = K_KEY * C_VOX
    body = functools.partial(_dfe_tgt_body, rows=rows)
    return pl.pallas_call(
        body,
        grid=(B, nblk),
        in_specs=[
            pl.BlockSpec((1, rows * NS_TGT, D_DFE), lambda i, r: (i, r, 0)),
            pl.BlockSpec((1, rows, D_DFE), lambda i, r: (i, r, 0)),
            pl.BlockSpec((D_DFE, D_DFE), lambda i, r: (0, 0)),
            pl.BlockSpec((1, D_DFE), lambda i, r: (0, 0)),
        ],
        out_specs=pl.BlockSpec((1, rows, D_DFE), lambda i, r: (i, r, 0)),
        out_shape=jax.ShapeDtypeStruct((B, nq, D_DFE), jnp.float32),
    )(g_tgt, sub_tgt, Wd2, bd2)


def _cpg_body(gs_ref, subs_ref, tdfe_ref, cand_ref, w2_ref, b2_ref, out_ref):
    gs = gs_ref[0]
    subs = subs_ref[0]
    h1 = jnp.maximum(gs.reshape(K_KEY, NS_SRC, D_DFE) - subs[:, None, :], 0.0)
    h1 = h1.reshape(K_KEY * NS_SRC, D_DFE)
    h2 = jnp.maximum(jnp.dot(h1, w2_ref[...], preferred_element_type=jnp.float32)
                     + b2_ref[...], 0.0)
    sdfe = jnp.max(h2.reshape(K_KEY, NS_SRC, D_DFE), axis=1)

    tdfe = tdfe_ref[0].reshape(K_KEY, C_VOX, D_DFE)
    sim = jnp.sum(sdfe[:, None, :] * tdfe, axis=-1)
    w = jnp.exp(sim - jnp.max(sim, axis=-1, keepdims=True))
    w = w / jnp.sum(w, axis=-1, keepdims=True)
    cand = cand_ref[0].reshape(K_KEY, C_VOX, 8)
    out_ref[0] = jnp.sum(w[:, :, None] * cand, axis=1)


def _run_cpg(g_src, sub_src, tgt_dfe, cand, Wd2, bd2):
    return pl.pallas_call(
        _cpg_body,
        grid=(B,),
        in_specs=[
            pl.BlockSpec((1, K_KEY * NS_SRC, D_DFE), lambda i: (i, 0, 0)),
            pl.BlockSpec((1, K_KEY, D_DFE), lambda i: (i, 0, 0)),
            pl.BlockSpec((1, K_KEY * C_VOX, D_DFE), lambda i: (i, 0, 0)),
            pl.BlockSpec((1, K_KEY * C_VOX, 8), lambda i: (i, 0, 0)),
            pl.BlockSpec((D_DFE, D_DFE), lambda i: (0, 0)),
            pl.BlockSpec((1, D_DFE), lambda i: (0, 0)),
        ],
        out_specs=pl.BlockSpec((1, K_KEY, 8), lambda i: (i, 0, 0)),
        out_shape=jax.ShapeDtypeStruct((B, K_KEY, 8), jnp.float32),
    )(g_src, sub_src, tgt_dfe, cand, Wd2, bd2)


def _pad_minor(x, width):
    pad = width - x.shape[-1]
    if pad == 0:
        return x
    cfg = [(0, 0)] * (x.ndim - 1) + [(0, pad)]
    return jnp.pad(x, cfg)


def kernel(src_pts, tgt_pts, R_init, t_init, W1, b1, W2, b2, W_wl, Wd1, bd1,
           Wd2, bd2):
    A = Wd1[:3]
    Ff = Wd1[3:]

    pts_all = jnp.concatenate([src_pts, tgt_pts], axis=0)
    x_all = _pad_minor(jnp.transpose(pts_all, (0, 2, 1)), 8)
    W1p = jnp.pad(W1, ((0, 2), (0, 0)))

    tables, scores = _run_k1(x_all, W1p, b1[None, :], W2, b2[None, :],
                             W_wl[:, None], jnp.pad(A, ((0, 5), (0, 0))), Ff)
    table_src = tables[:B].reshape(B * N, D_DFE)
    table_tgt = tables[B:].reshape(B * N, D_DFE)

    scores2 = scores[:B].reshape(B, 64, 128)
    xyz_src = x_all[:B]

    grid1 = jnp.arange(-1.0, 1.5, 1.0)
    offs = jnp.stack(jnp.meshgrid(grid1, grid1, grid1, indexing='ij'),
                     axis=-1).reshape(-1, 3)

    key_p, cand_p, sub_src, sub_tgt = _run_k2(
        scores2, xyz_src, jnp.pad(jnp.transpose(R_init), ((0, 5), (0, 5))),
        _pad_minor(jnp.transpose(t_init), 8), _pad_minor(offs, 8),
        jnp.pad(A, ((0, 5), (0, 0))), bd1[None, :])

    src_xyzT = src_pts[:, :3, :]
    tgt_xyzT = tgt_pts[:, :3, :]

    pidx = _run_knn(key_p, src_xyzT, NS_SRC, K_KEY, 1)
    tidx = _run_knn(cand_p, tgt_xyzT, NS_TGT, 216, 8)

    g_src, g_tgt = _run_sc_gather(table_src, table_tgt,
                                  pidx.reshape(-1), tidx.reshape(-1))
    g_src = g_src.reshape(B, K_KEY * NS_SRC, D_DFE)
    g_tgt = g_tgt.reshape(B, K_KEY * C_VOX * NS_TGT, D_DFE)

    tgt_dfe = _run_dfe_tgt(g_tgt, sub_tgt, Wd2, bd2[None, :], 216, 8)
    tgt_vcp = _run_cpg(g_src, sub_src, tgt_dfe, cand_p, Wd2, bd2[None, :])

    return key_p[:, :, :3], tgt_vcp[:, :, :3]
